# Initial kernel scaffold; baseline (speedup 1.0000x reference)
import jax
import jax.numpy as jnp
from jax.experimental import pallas as pl


def kernel(coords, features):
    raise NotImplementedError("write your pallas kernel here")


# speedup vs baseline: 29.5356x; 29.5356x over previous
import jax
import jax.numpy as jnp
from jax import lax
from jax.experimental import pallas as pl
from jax.experimental.pallas import tpu as pltpu
from jax.experimental.pallas import tpu_sc as plsc

_N = 16
_L = 4096
_M = 1024


_W = 128


def _fps_kernel(cx_ref, cy_ref, cz_ref,
                gidx_ref, ox_ref, oy_ref, oz_ref,
                dist_ref, lxr, lyr, lzr):
    j = pl.program_id(0)
    iota = lax.broadcasted_iota(jnp.int32, (_N, _L), 1)
    lane = lax.broadcasted_iota(jnp.int32, (_N, _W), 1)
    base = lax.broadcasted_iota(jnp.int32, (_N, 1), 0) * _L
    is0 = j == 0

    @pl.when(is0)
    def _init():
        dist_ref[...] = jnp.full((_N, _L), jnp.inf, jnp.float32)
        lxr[...] = cx_ref[:, 0:1]
        lyr[...] = cy_ref[:, 0:1]
        lzr[...] = cz_ref[:, 0:1]

    lx0 = lxr[...]
    ly0 = lyr[...]
    lz0 = lzr[...]
    seed = is0 & (lane == 0)
    gbuf0 = jnp.where(seed, base, 0)
    xbuf0 = jnp.where(seed, lx0, 0.0)
    ybuf0 = jnp.where(seed, ly0, 0.0)
    zbuf0 = jnp.where(seed, lz0, 0.0)

    def body(k, carry):
        gbuf, xbuf, ybuf, zbuf, lx, ly, lz = carry
        cx = cx_ref[...]
        cy = cy_ref[...]
        cz = cz_ref[...]
        dx = cx - lx
        dy = cy - ly
        dz = cz - lz
        d = dx * dx + dy * dy + dz * dz
        dm = jnp.minimum(dist_ref[...], d)
        dist_ref[...] = dm
        mx = jnp.max(dm, axis=1, keepdims=True)
        nxt = jnp.min(jnp.where(dm == mx, iota, _L), axis=1, keepdims=True)
        oh = iota == nxt
        nlx = jnp.sum(jnp.where(oh, cx, 0.0), axis=1, keepdims=True)
        nly = jnp.sum(jnp.where(oh, cy, 0.0), axis=1, keepdims=True)
        nlz = jnp.sum(jnp.where(oh, cz, 0.0), axis=1, keepdims=True)
        m = lane == k
        return (jnp.where(m, nxt + base, gbuf),
                jnp.where(m, nlx, xbuf),
                jnp.where(m, nly, ybuf),
                jnp.where(m, nlz, zbuf),
                nlx, nly, nlz)

    start = jnp.where(is0, 1, 0)
    gbuf, xbuf, ybuf, zbuf, lx, ly, lz = lax.fori_loop(
        start, _W, body, (gbuf0, xbuf0, ybuf0, zbuf0, lx0, ly0, lz0))

    gidx_ref[...] = gbuf
    ox_ref[...] = xbuf
    oy_ref[...] = ybuf
    oz_ref[...] = zbuf
    lxr[...] = lx
    lyr[...] = ly
    lzr[...] = lz


def _fps(cx, cy, cz):
    out_block = pl.BlockSpec((_N, _W), lambda j: (0, j))
    return pl.pallas_call(
        _fps_kernel,
        grid=(_M // _W,),
        in_specs=[pl.BlockSpec((_N, _L), lambda j: (0, 0))] * 3,
        out_specs=[out_block] * 4,
        out_shape=[
            jax.ShapeDtypeStruct((_N, _M), jnp.int32),
            jax.ShapeDtypeStruct((_N, _M), jnp.float32),
            jax.ShapeDtypeStruct((_N, _M), jnp.float32),
            jax.ShapeDtypeStruct((_N, _M), jnp.float32),
        ],
        scratch_shapes=[pltpu.VMEM((_N, _L), jnp.float32),
                        pltpu.VMEM((_N, 1), jnp.float32),
                        pltpu.VMEM((_N, 1), jnp.float32),
                        pltpu.VMEM((_N, 1), jnp.float32)],
    )(cx, cy, cz)


def _sc_gather(table, idx):
    num_idx = idx.shape[1]
    depth = table.shape[1]
    window = 128
    mesh = plsc.VectorSubcoreMesh(core_axis_name="core",
                                  subcore_axis_name="subcore")

    @pl.kernel(out_type=jax.ShapeDtypeStruct((num_idx, depth), table.dtype),
               mesh=mesh)
    def gather_kernel(x_hbm, i_hbm, o_hbm):
        def body(i_vmem, o_vmem):
            pltpu.sync_copy(x_hbm.at[i_vmem.at[0]], o_vmem)

        pltpu.emit_pipeline(
            body,
            grid=(num_idx // window,),
            in_specs=[pl.BlockSpec((1, window), index_map=lambda i: (0, i))],
            out_specs=[pl.BlockSpec((window, depth),
                                    index_map=lambda i: (i, 0))],
            core_axis_name=("core", "subcore"),
            dimension_semantics=(pltpu.PARALLEL,),
        )(i_hbm, o_hbm)

    return gather_kernel(table, idx)


def kernel(coords, features):
    cx = coords[:, :, 0]
    cy = coords[:, :, 1]
    cz = coords[:, :, 2]
    gidx, ox, oy, oz = _fps(cx, cy, cz)
    coords_out = jnp.stack([ox, oy, oz], axis=-1)
    depth = features.shape[-1]
    feats_flat = features.reshape(_N * _L, depth)
    feats_pad = jnp.pad(feats_flat, ((0, 0), (0, 128 - depth)))
    feats_out = _sc_gather(feats_pad, gidx.reshape(1, _N * _M))
    feats_out = feats_out[:, :depth].reshape(_N, _M, depth)
    return coords_out, feats_out
